# Initial kernel scaffold; baseline (speedup 1.0000x reference)
import jax
import jax.numpy as jnp
from jax.experimental import pallas as pl


def kernel(x, table):
    raise NotImplementedError("write your pallas kernel here")


# speedup vs baseline: 3.7354x; 3.7354x over previous
import functools

import jax
import jax.numpy as jnp
from jax import lax
from jax.experimental import pallas as pl
from jax.experimental.pallas import tpu as pltpu
from jax.experimental.pallas import tpu_sc as plsc

_D = 64
_CHUNK = 128


def _build(B, b_per_w):
    n_chunks = b_per_w // _CHUNK
    mesh = plsc.VectorSubcoreMesh(core_axis_name="c", subcore_axis_name="s")

    @functools.partial(
        pl.kernel,
        mesh=mesh,
        out_type=jax.ShapeDtypeStruct((B, _D), jnp.float32),
        compiler_params=pltpu.CompilerParams(use_tc_tiling_on_sc=False),
        scratch_types=[
            pltpu.VMEM((_CHUNK,), jnp.int32),
            pltpu.VMEM((_CHUNK, _D), jnp.float32),
            pltpu.SemaphoreType.DMA,
        ],
    )
    def gather_kernel(table_hbm, idx_hbm, out_hbm, idx_v, rows_v, sem):
        wid = lax.axis_index("s") * 2 + lax.axis_index("c")
        base = wid * b_per_w

        def body(i, carry):
            off = base + i * _CHUNK
            pltpu.sync_copy(idx_hbm.at[pl.ds(off, _CHUNK)], idx_v)
            pltpu.async_copy(table_hbm.at[idx_v], rows_v, sem).wait()
            pltpu.sync_copy(rows_v, out_hbm.at[pl.ds(off, _CHUNK)])
            return carry

        lax.fori_loop(0, n_chunks, body, 0)

    return gather_kernel


@jax.jit
def kernel(x, table):
    B = x.shape[0] * x.shape[1]
    b_per_w = B // 32
    out = _build(B, b_per_w)(table, x.reshape(-1))
    return out.reshape(x.shape[0], x.shape[1], _D)


# speedup vs baseline: 4.7230x; 1.2644x over previous
import functools

import jax
import jax.numpy as jnp
from jax import lax
from jax.experimental import pallas as pl
from jax.experimental.pallas import tpu as pltpu
from jax.experimental.pallas import tpu_sc as plsc

_D = 64
_IV = 128
_K = 4
_S = _IV * _K


def _build(B, b_per_w):
    n_sc = b_per_w // _S
    mesh = plsc.VectorSubcoreMesh(core_axis_name="c", subcore_axis_name="s")

    @functools.partial(
        pl.kernel,
        mesh=mesh,
        out_type=jax.ShapeDtypeStruct((B, _D), jnp.float32),
        compiler_params=pltpu.CompilerParams(use_tc_tiling_on_sc=False),
        scratch_types=[
            pltpu.VMEM((2, _S), jnp.int32),
            pltpu.VMEM((2, _S, _D), jnp.float32),
            pltpu.SemaphoreType.DMA,
            pltpu.SemaphoreType.DMA,
            pltpu.SemaphoreType.DMA,
        ],
    )
    def gather_kernel(table_hbm, idx_hbm, out_hbm, idx_v, rows_v, isem,
                      gsem, osem):
        wid = lax.axis_index("s") * 2 + lax.axis_index("c")
        base = wid * b_per_w

        def load_idx(g, slot):
            pltpu.async_copy(
                idx_hbm.at[pl.ds(base + g * _S, _S)],
                idx_v.at[slot],
                isem,
            )

        def fire_gathers(slot):
            for j in range(_K):
                pltpu.async_copy(
                    table_hbm.at[idx_v.at[slot, pl.ds(j * _IV, _IV)]],
                    rows_v.at[slot, pl.ds(j * _IV, _IV)],
                    gsem,
                )

        def drain_gathers(slot):
            for j in range(_K):
                pltpu.make_async_copy(
                    table_hbm.at[idx_v.at[slot, pl.ds(j * _IV, _IV)]],
                    rows_v.at[slot, pl.ds(j * _IV, _IV)],
                    gsem,
                ).wait()

        def store_out(g, slot):
            pltpu.async_copy(
                rows_v.at[slot],
                out_hbm.at[pl.ds(base + g * _S, _S)],
                osem,
            )

        def drain_out(g, slot):
            pltpu.make_async_copy(
                rows_v.at[slot],
                out_hbm.at[pl.ds(base + g * _S, _S)],
                osem,
            ).wait()

        def drain_idx(g, slot):
            pltpu.make_async_copy(
                idx_hbm.at[pl.ds(base + g * _S, _S)],
                idx_v.at[slot],
                isem,
            ).wait()

        load_idx(0, 0)
        load_idx(1, 1)
        drain_idx(0, 0)
        fire_gathers(0)

        def body(g, carry):
            cur = lax.rem(g, 2)
            nxt = lax.rem(g + 1, 2)
            drain_gathers(cur)

            @pl.when(g > 0)
            def _():
                drain_out(g - 1, nxt)

            store_out(g, cur)

            @pl.when(g + 1 < n_sc)
            def _():
                drain_idx(g + 1, nxt)
                fire_gathers(nxt)

            @pl.when(g + 2 < n_sc)
            def _():
                load_idx(g + 2, cur)

            return carry

        lax.fori_loop(0, n_sc, body, 0)
        drain_out(n_sc - 1, lax.rem(n_sc - 1, 2))

    return gather_kernel


@jax.jit
def kernel(x, table):
    B = x.shape[0] * x.shape[1]
    b_per_w = B // 32
    out = _build(B, b_per_w)(table, x.reshape(-1))
    return out.reshape(x.shape[0], x.shape[1], _D)
